# baseline (speedup 1.0000x reference)
import jax
import jax.numpy as jnp
from jax.experimental import pallas as pl
from jax.experimental.pallas import tpu as pltpu

_GPB = 2


def _dot(a, b):
    return jax.lax.dot_general(
        a, b, (((1,), (0,)), ((), ())),
        preferred_element_type=jnp.float32)


def _graph_layer_body(x_ref, mask_ref, s_ref, we_ref, w0_ref, w1_ref,
                      wh1_ref, be_ref, bz_ref, br_ref, bh_ref, out_ref):
    n, d = x_ref.shape[1], x_ref.shape[2]
    bf16 = jnp.bfloat16
    We = we_ref[...]
    W0 = w0_ref[...]
    W1 = w1_ref[...]
    Wh1 = wh1_ref[...]
    be = be_ref[...]
    bzr = jnp.concatenate([bz_ref[...], br_ref[...]], axis=1)
    bh = bh_ref[...]

    M = mask_ref[...].reshape(_GPB * n, 1)
    S = [s_ref[g].astype(bf16) for g in range(_GPB)]

    X0 = x_ref[...].reshape(_GPB * n, d).astype(bf16)
    X = M * jax.nn.relu(_dot(X0, We) + be)

    for _ in range(2):
        Xb = X.astype(bf16)
        A = [_dot(S[g], Xb[g * n:(g + 1) * n]).astype(bf16)
             for g in range(_GPB)]
        Xn = []
        for c in range(_GPB):
            lo = c * n
            Ac = A[c]
            Xc = X[lo:lo + n]
            Xbc = Xb[lo:lo + n]
            Mc = M[lo:lo + n]
            G0 = _dot(Ac, W0)
            G1 = _dot(Xbc, W1)
            ZR = jax.nn.sigmoid(G0[:, :2 * d] + G1 + bzr)
            z = ZR[:, :d]
            r = ZR[:, d:]
            H1 = _dot((r * Xc).astype(bf16), Wh1)
            h = jax.nn.relu(Mc * (G0[:, 2 * d:] + H1 + bh))
            Xn.append(h * z + Xc * (1.0 - z))
        X = jnp.concatenate(Xn, axis=0)

    out_ref[...] = X.reshape(_GPB, n, d)


def kernel(x, mask, support, weights_encode, weights_z0, weights_z1,
           weights_r0, weights_r1, weights_h0, weights_h1, bias_encode,
           bias_z0, bias_z1, bias_r0, bias_r1, bias_h0, bias_h1):
    b, n, d = x.shape
    bf16 = jnp.bfloat16

    w0 = jnp.concatenate([weights_z0, weights_r0, weights_h0], axis=1).astype(bf16)
    w1 = jnp.concatenate([weights_z1, weights_r1], axis=1).astype(bf16)
    wh1 = weights_h1.astype(bf16)
    we = weights_encode.astype(bf16)
    be = bias_encode.reshape(1, d)
    bz = (bias_z0 + bias_z1).reshape(1, d)
    br = (bias_r0 + bias_r1).reshape(1, d)
    bh = (bias_h0 + bias_h1).reshape(1, d)

    batch_spec = lambda shape: pl.BlockSpec(shape, lambda i: (i, 0, 0))
    full_spec = lambda shape: pl.BlockSpec(shape, lambda i: (0, 0))

    return pl.pallas_call(
        _graph_layer_body,
        grid=(b // _GPB,),
        in_specs=[
            batch_spec((_GPB, n, d)),
            batch_spec((_GPB, n, 1)),
            batch_spec((_GPB, n, n)),
            full_spec((d, d)),
            full_spec((d, 3 * d)),
            full_spec((d, 2 * d)),
            full_spec((d, d)),
            full_spec((1, d)),
            full_spec((1, d)),
            full_spec((1, d)),
            full_spec((1, d)),
        ],
        out_specs=batch_spec((_GPB, n, d)),
        out_shape=jax.ShapeDtypeStruct((b, n, d), jnp.float32),
        compiler_params=pltpu.CompilerParams(
            dimension_semantics=("parallel",)),
    )(x, mask, support, we, w0, w1, wh1, be, bz, br, bh)


# speedup vs baseline: 1.0526x; 1.0526x over previous
import jax
import jax.numpy as jnp
from jax.experimental import pallas as pl
from jax.experimental.pallas import tpu as pltpu

_GPB = 4


def _dot(a, b):
    return jax.lax.dot_general(
        a, b, (((1,), (0,)), ((), ())),
        preferred_element_type=jnp.float32)


def _graph_layer_body(x_ref, mask_ref, s_ref, we_ref, w0_ref, w1_ref,
                      wh1_ref, be_ref, bz_ref, br_ref, bh_ref, out_ref):
    n, d = x_ref.shape[1], x_ref.shape[2]
    bf16 = jnp.bfloat16
    We = we_ref[...]
    W0 = w0_ref[...]
    W1 = w1_ref[...]
    Wh1 = wh1_ref[...]
    be = be_ref[...]
    bzr = jnp.concatenate([bz_ref[...], br_ref[...]], axis=1)
    bh = bh_ref[...]

    M = mask_ref[...].reshape(_GPB * n, 1)
    S = [s_ref[g].astype(bf16) for g in range(_GPB)]

    X0 = x_ref[...].reshape(_GPB * n, d).astype(bf16)
    X = M * jax.nn.relu(_dot(X0, We) + be)

    for _ in range(2):
        Xb = X.astype(bf16)
        A = [_dot(S[g], Xb[g * n:(g + 1) * n]).astype(bf16)
             for g in range(_GPB)]
        Xn = []
        for c in range(_GPB):
            lo = c * n
            Ac = A[c]
            Xc = X[lo:lo + n]
            Xbc = Xb[lo:lo + n]
            Mc = M[lo:lo + n]
            G0 = _dot(Ac, W0)
            G1 = _dot(Xbc, W1)
            ZR = jax.nn.sigmoid(G0[:, :2 * d] + G1 + bzr)
            z = ZR[:, :d]
            r = ZR[:, d:]
            H1 = _dot((r * Xc).astype(bf16), Wh1)
            h = jax.nn.relu(Mc * (G0[:, 2 * d:] + H1 + bh))
            Xn.append(h * z + Xc * (1.0 - z))
        X = jnp.concatenate(Xn, axis=0)

    out_ref[...] = X.reshape(_GPB, n, d)


def kernel(x, mask, support, weights_encode, weights_z0, weights_z1,
           weights_r0, weights_r1, weights_h0, weights_h1, bias_encode,
           bias_z0, bias_z1, bias_r0, bias_r1, bias_h0, bias_h1):
    b, n, d = x.shape
    bf16 = jnp.bfloat16

    w0 = jnp.concatenate([weights_z0, weights_r0, weights_h0], axis=1).astype(bf16)
    w1 = jnp.concatenate([weights_z1, weights_r1], axis=1).astype(bf16)
    wh1 = weights_h1.astype(bf16)
    we = weights_encode.astype(bf16)
    be = bias_encode.reshape(1, d)
    bz = (bias_z0 + bias_z1).reshape(1, d)
    br = (bias_r0 + bias_r1).reshape(1, d)
    bh = (bias_h0 + bias_h1).reshape(1, d)

    batch_spec = lambda shape: pl.BlockSpec(shape, lambda i: (i, 0, 0))
    full_spec = lambda shape: pl.BlockSpec(shape, lambda i: (0, 0))

    return pl.pallas_call(
        _graph_layer_body,
        grid=(b // _GPB,),
        in_specs=[
            batch_spec((_GPB, n, d)),
            batch_spec((_GPB, n, 1)),
            batch_spec((_GPB, n, n)),
            full_spec((d, d)),
            full_spec((d, 3 * d)),
            full_spec((d, 2 * d)),
            full_spec((d, d)),
            full_spec((1, d)),
            full_spec((1, d)),
            full_spec((1, d)),
            full_spec((1, d)),
        ],
        out_specs=batch_spec((_GPB, n, d)),
        out_shape=jax.ShapeDtypeStruct((b, n, d), jnp.float32),
        compiler_params=pltpu.CompilerParams(
            dimension_semantics=("parallel",)),
    )(x, mask, support, we, w0, w1, wh1, be, bz, br, bh)
